# baseline (speedup 1.0000x reference)
import functools

import jax
import jax.numpy as jnp
from jax import lax
from jax.experimental import pallas as pl
from jax.experimental.pallas import tpu as pltpu
from jax.experimental.pallas import tpu_sc as plsc

N = 10000
D = 128
E = 320000

NC = 2
NS = 16
NW = NC * NS
EB = 128
BLOCKS = 80
E_PAD = NW * BLOCKS * EB
E_PER_SC = E_PAD // NC
E_PER_TILE = E_PER_SC // NS
N_PAD = 10240
ROWS_PER_TILE = N_PAD // NS
STAGE_ROWS = 128
N_STAGE = ROWS_PER_TILE // STAGE_ROWS


def _deg_body(dst_hbm, ones_hbm, zeros_hbm, out_hbm, didx, crow, acc):
    c = lax.axis_index("c")
    s = lax.axis_index("s")
    pltpu.sync_copy(zeros_hbm, crow)
    for k in range(N_STAGE):
        pltpu.sync_copy(
            crow, acc.at[pl.ds(s * ROWS_PER_TILE + k * STAGE_ROWS, STAGE_ROWS)]
        )
    pltpu.sync_copy(ones_hbm, crow)
    plsc.subcore_barrier()

    tile_base = c * E_PER_SC + s * E_PER_TILE

    def body(i, carry):
        base = tile_base + i * EB
        pltpu.sync_copy(dst_hbm.at[pl.ds(base, EB)], didx)
        pltpu.sync_copy(crow, acc.at[didx], add=True)
        return carry

    lax.fori_loop(0, BLOCKS, body, 0)
    plsc.subcore_barrier()
    for k in range(N_STAGE):
        r0 = s * ROWS_PER_TILE + k * STAGE_ROWS
        pltpu.sync_copy(acc.at[pl.ds(r0, STAGE_ROWS)], crow)
        pltpu.sync_copy(crow, out_hbm.at[pl.ds(c * N_PAD + r0, STAGE_ROWS)])


@functools.cache
def _deg_call():
    return pl.kernel(
        _deg_body,
        mesh=plsc.VectorSubcoreMesh(core_axis_name="c", subcore_axis_name="s"),
        out_type=jax.ShapeDtypeStruct((2 * N_PAD, D), jnp.float32),
        scratch_types=[
            pltpu.VMEM((EB,), jnp.int32),
            pltpu.VMEM((EB, D), jnp.float32),
            pltpu.VMEM_SHARED((N_PAD, D), jnp.float32),
        ],
    )


def _agg_body(hp_hbm, pairs_hbm, zeros_hbm, out_hbm,
              pidx0, pidx1, rows0, rows1, acc, sem0, sem1):
    c = lax.axis_index("c")
    s = lax.axis_index("s")

    @pl.when(c == 0)
    def _():
        pltpu.sync_copy(zeros_hbm, rows0)
        for k in range(N_STAGE):
            pltpu.sync_copy(
                rows0,
                acc.at[pl.ds(s * ROWS_PER_TILE + k * STAGE_ROWS, STAGE_ROWS)],
            )
        plsc.subcore_barrier()

        tile_blk0 = s * (2 * BLOCKS)

        pltpu.sync_copy(pairs_hbm.at[pl.ds(2 * tile_blk0, 2)], pidx0)
        pltpu.async_copy(hp_hbm.at[pidx0.at[0]], rows0, sem0)

        def body(g, carry):
            b0 = tile_blk0 + 2 * g
            pltpu.sync_copy(pairs_hbm.at[pl.ds(2 * (b0 + 1), 2)], pidx1)
            pltpu.async_copy(hp_hbm.at[pidx1.at[0]], rows1, sem1)
            pltpu.make_async_copy(hp_hbm.at[pidx0.at[0]], rows0, sem0).wait()
            pltpu.sync_copy(rows0, acc.at[pidx0.at[1]], add=True)
            pltpu.sync_copy(pairs_hbm.at[pl.ds(2 * (b0 + 2), 2)], pidx0)
            pltpu.async_copy(hp_hbm.at[pidx0.at[0]], rows0, sem0)
            pltpu.make_async_copy(hp_hbm.at[pidx1.at[0]], rows1, sem1).wait()
            pltpu.sync_copy(rows1, acc.at[pidx1.at[1]], add=True)
            return carry

        lax.fori_loop(0, BLOCKS, body, 0)
        pltpu.make_async_copy(hp_hbm.at[pidx0.at[0]], rows0, sem0).wait()
        plsc.subcore_barrier()
        for k in range(N_STAGE):
            r0 = s * ROWS_PER_TILE + k * STAGE_ROWS
            pltpu.sync_copy(acc.at[pl.ds(r0, STAGE_ROWS)], rows0)
            pltpu.sync_copy(rows0, out_hbm.at[pl.ds(r0, STAGE_ROWS)])


@functools.cache
def _agg_call():
    return pl.kernel(
        _agg_body,
        mesh=plsc.VectorSubcoreMesh(core_axis_name="c", subcore_axis_name="s"),
        out_type=jax.ShapeDtypeStruct((N_PAD, D), jnp.float32),
        scratch_types=[
            pltpu.VMEM((2, EB), jnp.int32),
            pltpu.VMEM((2, EB), jnp.int32),
            pltpu.VMEM((EB, D), jnp.float32),
            pltpu.VMEM((EB, D), jnp.float32),
            pltpu.VMEM_SHARED((N_PAD, D), jnp.float32),
            pltpu.SemaphoreType.DMA,
            pltpu.SemaphoreType.DMA,
        ],
    )


R_BLK = 2000
GRID = N // R_BLK


def _tc1_body(x_ref, w_ref, d0_ref, d1_ref, o_ref, dis_ref):
    deg = d0_ref[:, 0:1] + d1_ref[:, 0:1] + 1.0
    dis = lax.rsqrt(deg)
    h = jnp.dot(x_ref[:], w_ref[:], preferred_element_type=jnp.float32)
    o_ref[:] = h * dis
    dis_ref[:] = jnp.broadcast_to(dis, (R_BLK, D))


def _tc2_body(p0_ref, hp_ref, dis_ref, b_ref, w_ref, o_ref):
    dis = dis_ref[:]
    z = dis * (p0_ref[:] + hp_ref[:]) + b_ref[:]
    h = jnp.maximum(z, 0.0)
    h2 = jnp.dot(h, w_ref[:], preferred_element_type=jnp.float32)
    o_ref[:] = h2 * dis


def _tc3_body(p0_ref, hp_ref, dis_ref, b_ref, o_ref):
    o_ref[:] = dis_ref[:] * (p0_ref[:] + hp_ref[:]) + b_ref[:]


_row_spec = pl.BlockSpec((R_BLK, D), lambda i: (i, 0))
_w_spec = pl.BlockSpec((D, D), lambda i: (0, 0))
_b_spec = pl.BlockSpec((1, D), lambda i: (0, 0))
_out_sd = jax.ShapeDtypeStruct((N, D), jnp.float32)

_tc1 = pl.pallas_call(
    _tc1_body, grid=(GRID,),
    in_specs=[_row_spec, _w_spec, _row_spec, _row_spec],
    out_specs=[_row_spec, _row_spec], out_shape=[_out_sd, _out_sd])

_tc2 = pl.pallas_call(
    _tc2_body, grid=(GRID,),
    in_specs=[_row_spec, _row_spec, _row_spec, _b_spec, _w_spec],
    out_specs=_row_spec, out_shape=_out_sd)

_tc3 = pl.pallas_call(
    _tc3_body, grid=(GRID,),
    in_specs=[_row_spec, _row_spec, _row_spec, _b_spec],
    out_specs=_row_spec, out_shape=_out_sd)


def kernel(x, edge_index, W1, b1, W2, b2):
    src = edge_index[0]
    dst = edge_index[1]
    e_alloc = E_PAD + EB
    src_p = jnp.concatenate([src, jnp.zeros((e_alloc - E,), jnp.int32)])
    dst_p = jnp.concatenate([dst, jnp.full((e_alloc - E,), N, jnp.int32)])
    pairs = jnp.stack(
        [src_p.reshape(-1, EB), dst_p.reshape(-1, EB)], axis=1
    ).reshape(-1, EB)

    ones128 = jnp.zeros((EB, D), jnp.float32).at[:, 0].set(1.0)
    zeros128 = jnp.zeros((EB, D), jnp.float32)

    degc = _deg_call()(dst_p[:E_PAD], ones128, zeros128)

    b1r = b1.reshape(1, D)
    b2r = b2.reshape(1, D)

    h1p, dis = _tc1(x, W1, degc[:N], degc[N_PAD:N_PAD + N])
    s1 = _agg_call()(h1p, pairs, zeros128)
    h2p = _tc2(s1[:N], h1p, dis, b1r, W2)
    s2 = _agg_call()(h2p, pairs, zeros128)
    return _tc3(s2[:N], h2p, dis, b2r)


# speedup vs baseline: 1.1579x; 1.1579x over previous
import functools

import jax
import jax.numpy as jnp
from jax import lax
from jax.experimental import pallas as pl
from jax.experimental.pallas import tpu as pltpu
from jax.experimental.pallas import tpu_sc as plsc

N = 10000
D = 128
E = 320000

NC = 2
NS = 16
NW = NC * NS
EB = 128
BLOCKS = 80
E_PAD = NW * BLOCKS * EB
E_PER_SC = E_PAD // NC
E_PER_TILE = E_PER_SC // NS
N_PAD = 10240
ROWS_PER_TILE = N_PAD // NS
STAGE_ROWS = 128
N_STAGE = ROWS_PER_TILE // STAGE_ROWS

AEB = 64
NBUF = 4
ABLK0 = 160
ABLK1 = 320 - ABLK0
A_N_STAGE = ROWS_PER_TILE // AEB


def _deg_body(dst_hbm, ones_hbm, zeros_hbm, out_hbm, didx, crow, acc):
    c = lax.axis_index("c")
    s = lax.axis_index("s")
    pltpu.sync_copy(zeros_hbm, crow)
    for k in range(N_STAGE):
        pltpu.sync_copy(
            crow, acc.at[pl.ds(s * ROWS_PER_TILE + k * STAGE_ROWS, STAGE_ROWS)]
        )
    pltpu.sync_copy(ones_hbm, crow)
    plsc.subcore_barrier()

    tile_base = c * E_PER_SC + s * E_PER_TILE

    def body(i, carry):
        base = tile_base + i * EB
        pltpu.sync_copy(dst_hbm.at[pl.ds(base, EB)], didx)
        pltpu.sync_copy(crow, acc.at[didx], add=True)
        return carry

    lax.fori_loop(0, BLOCKS, body, 0)
    plsc.subcore_barrier()
    for k in range(N_STAGE):
        r0 = s * ROWS_PER_TILE + k * STAGE_ROWS
        pltpu.sync_copy(acc.at[pl.ds(r0, STAGE_ROWS)], crow)
        pltpu.sync_copy(crow, out_hbm.at[pl.ds(c * N_PAD + r0, STAGE_ROWS)])


@functools.cache
def _deg_call():
    return pl.kernel(
        _deg_body,
        mesh=plsc.VectorSubcoreMesh(core_axis_name="c", subcore_axis_name="s"),
        out_type=jax.ShapeDtypeStruct((2 * N_PAD, D), jnp.float32),
        scratch_types=[
            pltpu.VMEM((EB,), jnp.int32),
            pltpu.VMEM((EB, D), jnp.float32),
            pltpu.VMEM_SHARED((N_PAD, D), jnp.float32),
        ],
    )


def _agg_body(hp_hbm, pairs_hbm, zeros_hbm, out_hbm, *sc):
    pidx = sc[:NBUF]
    rows = sc[NBUF:2 * NBUF]
    acc = sc[2 * NBUF]
    sems = sc[2 * NBUF + 1:]
    c = lax.axis_index("c")
    s = lax.axis_index("s")

    pltpu.sync_copy(zeros_hbm, rows[0])
    for k in range(A_N_STAGE):
        pltpu.sync_copy(
            rows[0], acc.at[pl.ds(s * ROWS_PER_TILE + k * AEB, AEB)]
        )
    plsc.subcore_barrier()

    tile_blk0 = jnp.where(c == 0, s * ABLK0, NS * ABLK0 + s * ABLK1)
    n_outer = jnp.where(c == 0, ABLK0 // NBUF, ABLK1 // NBUF)

    def fire(blk, b):
        pltpu.sync_copy(pairs_hbm.at[pl.ds(2 * blk, 2)], pidx[b])
        pltpu.async_copy(hp_hbm.at[pidx[b].at[0]], rows[b], sems[b])

    def drain(b):
        pltpu.make_async_copy(hp_hbm.at[pidx[b].at[0]], rows[b], sems[b]).wait()

    for b in range(NBUF - 1):
        fire(tile_blk0 + b, b)

    def body(g, carry):
        i0 = tile_blk0 + NBUF * g
        for b in range(NBUF):
            fire(i0 + b + NBUF - 1, (b + NBUF - 1) % NBUF)
            drain(b)
            pltpu.sync_copy(rows[b], acc.at[pidx[b].at[1]], add=True)
        return carry

    lax.fori_loop(0, n_outer, body, 0)
    for b in range(NBUF - 1):
        drain(b)
    plsc.subcore_barrier()
    for k in range(A_N_STAGE):
        r0 = s * ROWS_PER_TILE + k * AEB
        pltpu.sync_copy(acc.at[pl.ds(r0, AEB)], rows[0])
        pltpu.sync_copy(rows[0], out_hbm.at[pl.ds(c * N_PAD + r0, AEB)])


@functools.cache
def _agg_call():
    return pl.kernel(
        _agg_body,
        mesh=plsc.VectorSubcoreMesh(core_axis_name="c", subcore_axis_name="s"),
        out_type=jax.ShapeDtypeStruct((2 * N_PAD, D), jnp.float32),
        scratch_types=(
            [pltpu.VMEM((2, AEB), jnp.int32) for _ in range(NBUF)]
            + [pltpu.VMEM((AEB, D), jnp.float32) for _ in range(NBUF)]
            + [pltpu.VMEM_SHARED((N_PAD, D), jnp.float32)]
            + [pltpu.SemaphoreType.DMA for _ in range(NBUF)]
        ),
    )


R_BLK = 2000
GRID = N // R_BLK


def _tc1_body(x_ref, w_ref, d0_ref, d1_ref, o_ref, dis_ref):
    deg = d0_ref[:, 0:1] + d1_ref[:, 0:1] + 1.0
    dis = lax.rsqrt(deg)
    h = jnp.dot(x_ref[:], w_ref[:], preferred_element_type=jnp.float32)
    o_ref[:] = h * dis
    dis_ref[:] = jnp.broadcast_to(dis, (R_BLK, D))


def _tc2_body(p0_ref, p1_ref, hp_ref, dis_ref, b_ref, w_ref, o_ref):
    dis = dis_ref[:]
    z = dis * (p0_ref[:] + p1_ref[:] + hp_ref[:]) + b_ref[:]
    h = jnp.maximum(z, 0.0)
    h2 = jnp.dot(h, w_ref[:], preferred_element_type=jnp.float32)
    o_ref[:] = h2 * dis


def _tc3_body(p0_ref, p1_ref, hp_ref, dis_ref, b_ref, o_ref):
    o_ref[:] = dis_ref[:] * (p0_ref[:] + p1_ref[:] + hp_ref[:]) + b_ref[:]


_row_spec = pl.BlockSpec((R_BLK, D), lambda i: (i, 0))
_w_spec = pl.BlockSpec((D, D), lambda i: (0, 0))
_b_spec = pl.BlockSpec((1, D), lambda i: (0, 0))
_out_sd = jax.ShapeDtypeStruct((N, D), jnp.float32)

_tc1 = pl.pallas_call(
    _tc1_body, grid=(GRID,),
    in_specs=[_row_spec, _w_spec, _row_spec, _row_spec],
    out_specs=[_row_spec, _row_spec], out_shape=[_out_sd, _out_sd])

_tc2 = pl.pallas_call(
    _tc2_body, grid=(GRID,),
    in_specs=[_row_spec, _row_spec, _row_spec, _row_spec, _b_spec, _w_spec],
    out_specs=_row_spec, out_shape=_out_sd)

_tc3 = pl.pallas_call(
    _tc3_body, grid=(GRID,),
    in_specs=[_row_spec, _row_spec, _row_spec, _row_spec, _b_spec],
    out_specs=_row_spec, out_shape=_out_sd)


def kernel(x, edge_index, W1, b1, W2, b2):
    src = edge_index[0]
    dst = edge_index[1]
    e_alloc = E_PAD + (NBUF - 1) * AEB
    src_p = jnp.concatenate([src, jnp.zeros((e_alloc - E,), jnp.int32)])
    dst_p = jnp.concatenate([dst, jnp.full((e_alloc - E,), N, jnp.int32)])
    pairs = jnp.stack(
        [src_p.reshape(-1, AEB), dst_p.reshape(-1, AEB)], axis=1
    ).reshape(-1, AEB)

    ones128 = jnp.zeros((EB, D), jnp.float32).at[:, 0].set(1.0)
    zeros128 = jnp.zeros((EB, D), jnp.float32)
    zeros_a = jnp.zeros((AEB, D), jnp.float32)

    degc = _deg_call()(dst_p[:E_PAD], ones128, zeros128)

    b1r = b1.reshape(1, D)
    b2r = b2.reshape(1, D)

    h1p, dis = _tc1(x, W1, degc[:N], degc[N_PAD:N_PAD + N])
    s1 = _agg_call()(h1p, pairs, zeros_a)
    h2p = _tc2(s1[:N], s1[N_PAD:N_PAD + N], h1p, dis, b1r, W2)
    s2 = _agg_call()(h2p, pairs, zeros_a)
    return _tc3(s2[:N], s2[N_PAD:N_PAD + N], h2p, dis, b2r)


# speedup vs baseline: 1.2353x; 1.0668x over previous
import functools

import jax
import jax.numpy as jnp
from jax import lax
from jax.experimental import pallas as pl
from jax.experimental.pallas import tpu as pltpu
from jax.experimental.pallas import tpu_sc as plsc

N = 10000
D = 128
E = 320000

NC = 2
NS = 16
NW = NC * NS
EB = 128
BLOCKS = 80
E_PAD = NW * BLOCKS * EB
E_PER_SC = E_PAD // NC
E_PER_TILE = E_PER_SC // NS
N_PAD = 10240
ROWS_PER_TILE = N_PAD // NS
STAGE_ROWS = 128
N_STAGE = ROWS_PER_TILE // STAGE_ROWS

AEB = 128
NBUF = 2
ABLK0 = 112
ABLK1 = 160 - ABLK0
A_N_STAGE = ROWS_PER_TILE // AEB


def _deg_body(dst_hbm, ones_hbm, zeros_hbm, out_hbm, didx, crow, acc):
    c = lax.axis_index("c")
    s = lax.axis_index("s")
    pltpu.sync_copy(zeros_hbm, crow)
    for k in range(N_STAGE):
        pltpu.sync_copy(
            crow, acc.at[pl.ds(s * ROWS_PER_TILE + k * STAGE_ROWS, STAGE_ROWS)]
        )
    pltpu.sync_copy(ones_hbm, crow)
    plsc.subcore_barrier()

    tile_base = c * E_PER_SC + s * E_PER_TILE

    def body(i, carry):
        base = tile_base + i * EB
        pltpu.sync_copy(dst_hbm.at[pl.ds(base, EB)], didx)
        pltpu.sync_copy(crow, acc.at[didx], add=True)
        return carry

    lax.fori_loop(0, BLOCKS, body, 0)
    plsc.subcore_barrier()
    for k in range(N_STAGE):
        r0 = s * ROWS_PER_TILE + k * STAGE_ROWS
        pltpu.sync_copy(acc.at[pl.ds(r0, STAGE_ROWS)], crow)
        pltpu.sync_copy(crow, out_hbm.at[pl.ds(c * N_PAD + r0, STAGE_ROWS)])


@functools.cache
def _deg_call():
    return pl.kernel(
        _deg_body,
        mesh=plsc.VectorSubcoreMesh(core_axis_name="c", subcore_axis_name="s"),
        out_type=jax.ShapeDtypeStruct((2 * N_PAD, D), jnp.float32),
        scratch_types=[
            pltpu.VMEM((EB,), jnp.int32),
            pltpu.VMEM((EB, D), jnp.float32),
            pltpu.VMEM_SHARED((N_PAD, D), jnp.float32),
        ],
    )


def _agg_body(hp_hbm, pairs_hbm, zeros_hbm, out_hbm, *sc):
    pidx = sc[:NBUF]
    rows = sc[NBUF:2 * NBUF]
    acc = sc[2 * NBUF]
    sems = sc[2 * NBUF + 1:]
    c = lax.axis_index("c")
    s = lax.axis_index("s")

    pltpu.sync_copy(zeros_hbm, rows[0])
    for k in range(A_N_STAGE):
        pltpu.sync_copy(
            rows[0], acc.at[pl.ds(s * ROWS_PER_TILE + k * AEB, AEB)]
        )
    plsc.subcore_barrier()

    tile_blk0 = jnp.where(c == 0, s * ABLK0, NS * ABLK0 + s * ABLK1)
    n_outer = jnp.where(c == 0, ABLK0 // NBUF, ABLK1 // NBUF)

    def fire(blk, b):
        pltpu.sync_copy(pairs_hbm.at[pl.ds(2 * blk, 2)], pidx[b])
        pltpu.async_copy(hp_hbm.at[pidx[b].at[0]], rows[b], sems[b])

    def drain(b):
        pltpu.make_async_copy(hp_hbm.at[pidx[b].at[0]], rows[b], sems[b]).wait()

    for b in range(NBUF - 1):
        fire(tile_blk0 + b, b)

    def body(g, carry):
        i0 = tile_blk0 + NBUF * g
        for b in range(NBUF):
            fire(i0 + b + NBUF - 1, (b + NBUF - 1) % NBUF)
            drain(b)
            pltpu.sync_copy(rows[b], acc.at[pidx[b].at[1]], add=True)
        return carry

    lax.fori_loop(0, n_outer, body, 0)
    for b in range(NBUF - 1):
        drain(b)
    plsc.subcore_barrier()
    for k in range(A_N_STAGE):
        r0 = s * ROWS_PER_TILE + k * AEB
        pltpu.sync_copy(acc.at[pl.ds(r0, AEB)], rows[0])
        pltpu.sync_copy(rows[0], out_hbm.at[pl.ds(c * N_PAD + r0, AEB)])


@functools.cache
def _agg_call():
    return pl.kernel(
        _agg_body,
        mesh=plsc.VectorSubcoreMesh(core_axis_name="c", subcore_axis_name="s"),
        out_type=jax.ShapeDtypeStruct((2 * N_PAD, D), jnp.float32),
        scratch_types=(
            [pltpu.VMEM((2, AEB), jnp.int32) for _ in range(NBUF)]
            + [pltpu.VMEM((AEB, D), jnp.float32) for _ in range(NBUF)]
            + [pltpu.VMEM_SHARED((N_PAD, D), jnp.float32)]
            + [pltpu.SemaphoreType.DMA for _ in range(NBUF)]
        ),
    )


R_BLK = 2000
GRID = N // R_BLK


def _tc1_body(x_ref, w_ref, d0_ref, d1_ref, o_ref, dis_ref):
    deg = d0_ref[:, 0:1] + d1_ref[:, 0:1] + 1.0
    dis = lax.rsqrt(deg)
    h = jnp.dot(x_ref[:], w_ref[:], preferred_element_type=jnp.float32)
    o_ref[:] = h * dis
    dis_ref[:] = jnp.broadcast_to(dis, (R_BLK, D))


def _tc2_body(p0_ref, p1_ref, hp_ref, dis_ref, b_ref, w_ref, o_ref):
    dis = dis_ref[:]
    z = dis * (p0_ref[:] + p1_ref[:] + hp_ref[:]) + b_ref[:]
    h = jnp.maximum(z, 0.0)
    h2 = jnp.dot(h, w_ref[:], preferred_element_type=jnp.float32)
    o_ref[:] = h2 * dis


def _tc3_body(p0_ref, p1_ref, hp_ref, dis_ref, b_ref, o_ref):
    o_ref[:] = dis_ref[:] * (p0_ref[:] + p1_ref[:] + hp_ref[:]) + b_ref[:]


_row_spec = pl.BlockSpec((R_BLK, D), lambda i: (i, 0))
_w_spec = pl.BlockSpec((D, D), lambda i: (0, 0))
_b_spec = pl.BlockSpec((1, D), lambda i: (0, 0))
_out_sd = jax.ShapeDtypeStruct((N, D), jnp.float32)

_tc1 = pl.pallas_call(
    _tc1_body, grid=(GRID,),
    in_specs=[_row_spec, _w_spec, _row_spec, _row_spec],
    out_specs=[_row_spec, _row_spec], out_shape=[_out_sd, _out_sd])

_tc2 = pl.pallas_call(
    _tc2_body, grid=(GRID,),
    in_specs=[_row_spec, _row_spec, _row_spec, _row_spec, _b_spec, _w_spec],
    out_specs=_row_spec, out_shape=_out_sd)

_tc3 = pl.pallas_call(
    _tc3_body, grid=(GRID,),
    in_specs=[_row_spec, _row_spec, _row_spec, _row_spec, _b_spec],
    out_specs=_row_spec, out_shape=_out_sd)


def kernel(x, edge_index, W1, b1, W2, b2):
    src = edge_index[0]
    dst = edge_index[1]
    e_alloc = E_PAD + (NBUF - 1) * AEB
    src_p = jnp.concatenate([src, jnp.zeros((e_alloc - E,), jnp.int32)])
    dst_p = jnp.concatenate([dst, jnp.full((e_alloc - E,), N, jnp.int32)])
    pairs = jnp.stack(
        [src_p.reshape(-1, AEB), dst_p.reshape(-1, AEB)], axis=1
    ).reshape(-1, AEB)

    ones128 = jnp.zeros((EB, D), jnp.float32).at[:, 0].set(1.0)
    zeros128 = jnp.zeros((EB, D), jnp.float32)
    zeros_a = jnp.zeros((AEB, D), jnp.float32)

    degc = _deg_call()(dst_p[:E_PAD], ones128, zeros128)

    b1r = b1.reshape(1, D)
    b2r = b2.reshape(1, D)

    h1p, dis = _tc1(x, W1, degc[:N], degc[N_PAD:N_PAD + N])
    s1 = _agg_call()(h1p, pairs, zeros_a)
    h2p = _tc2(s1[:N], s1[N_PAD:N_PAD + N], h1p, dis, b1r, W2)
    s2 = _agg_call()(h2p, pairs, zeros_a)
    return _tc3(s2[:N], s2[N_PAD:N_PAD + N], h2p, dis, b2r)


# speedup vs baseline: 1.2433x; 1.0064x over previous
import functools

import jax
import jax.numpy as jnp
from jax import lax
from jax.experimental import pallas as pl
from jax.experimental.pallas import tpu as pltpu
from jax.experimental.pallas import tpu_sc as plsc

N = 10000
D = 128
E = 320000

NC = 2
NS = 16
NW = NC * NS
EB = 128
BLOCKS = 80
E_PAD = NW * BLOCKS * EB
E_PER_SC = E_PAD // NC
E_PER_TILE = E_PER_SC // NS
N_PAD = 10240
ROWS_PER_TILE = N_PAD // NS
STAGE_ROWS = 128
N_STAGE = ROWS_PER_TILE // STAGE_ROWS

AEB = 128
NBUF = 2
ABLK0 = 120
ABLK1 = 160 - ABLK0
A_N_STAGE = ROWS_PER_TILE // AEB


def _deg_body(dst_hbm, ones_hbm, zeros_hbm, out_hbm, didx, crow, acc):
    c = lax.axis_index("c")
    s = lax.axis_index("s")
    pltpu.sync_copy(zeros_hbm, crow)
    for k in range(N_STAGE):
        pltpu.sync_copy(
            crow, acc.at[pl.ds(s * ROWS_PER_TILE + k * STAGE_ROWS, STAGE_ROWS)]
        )
    pltpu.sync_copy(ones_hbm, crow)
    plsc.subcore_barrier()

    tile_base = c * E_PER_SC + s * E_PER_TILE

    def body(i, carry):
        base = tile_base + i * EB
        pltpu.sync_copy(dst_hbm.at[pl.ds(base, EB)], didx)
        pltpu.sync_copy(crow, acc.at[didx], add=True)
        return carry

    lax.fori_loop(0, BLOCKS, body, 0)
    plsc.subcore_barrier()
    for k in range(N_STAGE):
        r0 = s * ROWS_PER_TILE + k * STAGE_ROWS
        pltpu.sync_copy(acc.at[pl.ds(r0, STAGE_ROWS)], crow)
        pltpu.sync_copy(crow, out_hbm.at[pl.ds(c * N_PAD + r0, STAGE_ROWS)])


@functools.cache
def _deg_call():
    return pl.kernel(
        _deg_body,
        mesh=plsc.VectorSubcoreMesh(core_axis_name="c", subcore_axis_name="s"),
        out_type=jax.ShapeDtypeStruct((2 * N_PAD, D), jnp.float32),
        scratch_types=[
            pltpu.VMEM((EB,), jnp.int32),
            pltpu.VMEM((EB, D), jnp.float32),
            pltpu.VMEM_SHARED((N_PAD, D), jnp.float32),
        ],
    )


def _agg_body(hp_hbm, pairs_hbm, zeros_hbm, out_hbm, *sc):
    pidx = sc[:NBUF]
    rows = sc[NBUF:2 * NBUF]
    acc = sc[2 * NBUF]
    sems = sc[2 * NBUF + 1:]
    c = lax.axis_index("c")
    s = lax.axis_index("s")

    pltpu.sync_copy(zeros_hbm, rows[0])
    for k in range(A_N_STAGE):
        pltpu.sync_copy(
            rows[0], acc.at[pl.ds(s * ROWS_PER_TILE + k * AEB, AEB)]
        )
    plsc.subcore_barrier()

    tile_blk0 = jnp.where(c == 0, s * ABLK0, NS * ABLK0 + s * ABLK1)
    n_outer = jnp.where(c == 0, ABLK0 // NBUF, ABLK1 // NBUF)

    def fire(blk, b):
        pltpu.sync_copy(pairs_hbm.at[pl.ds(2 * blk, 2)], pidx[b])
        pltpu.async_copy(hp_hbm.at[pidx[b].at[0]], rows[b], sems[b])

    def drain(b):
        pltpu.make_async_copy(hp_hbm.at[pidx[b].at[0]], rows[b], sems[b]).wait()

    for b in range(NBUF - 1):
        fire(tile_blk0 + b, b)

    def body(g, carry):
        i0 = tile_blk0 + NBUF * g
        for b in range(NBUF):
            fire(i0 + b + NBUF - 1, (b + NBUF - 1) % NBUF)
            drain(b)
            pltpu.sync_copy(rows[b], acc.at[pidx[b].at[1]], add=True)
        return carry

    lax.fori_loop(0, n_outer, body, 0)
    for b in range(NBUF - 1):
        drain(b)
    plsc.subcore_barrier()
    for k in range(A_N_STAGE):
        r0 = s * ROWS_PER_TILE + k * AEB
        pltpu.sync_copy(acc.at[pl.ds(r0, AEB)], rows[0])
        pltpu.sync_copy(rows[0], out_hbm.at[pl.ds(c * N_PAD + r0, AEB)])


@functools.cache
def _agg_call():
    return pl.kernel(
        _agg_body,
        mesh=plsc.VectorSubcoreMesh(core_axis_name="c", subcore_axis_name="s"),
        out_type=jax.ShapeDtypeStruct((2 * N_PAD, D), jnp.float32),
        scratch_types=(
            [pltpu.VMEM((2, AEB), jnp.int32) for _ in range(NBUF)]
            + [pltpu.VMEM((AEB, D), jnp.float32) for _ in range(NBUF)]
            + [pltpu.VMEM_SHARED((N_PAD, D), jnp.float32)]
            + [pltpu.SemaphoreType.DMA for _ in range(NBUF)]
        ),
    )


R_BLK = 2000
GRID = N // R_BLK


def _tc1_body(x_ref, w_ref, d0_ref, d1_ref, o_ref, dis_ref):
    deg = d0_ref[:, 0:1] + d1_ref[:, 0:1] + 1.0
    dis = lax.rsqrt(deg)
    h = jnp.dot(x_ref[:], w_ref[:], preferred_element_type=jnp.float32)
    o_ref[:] = h * dis
    dis_ref[:] = jnp.broadcast_to(dis, (R_BLK, D))


def _tc2_body(p0_ref, p1_ref, hp_ref, dis_ref, b_ref, w_ref, o_ref):
    dis = dis_ref[:]
    z = dis * (p0_ref[:] + p1_ref[:] + hp_ref[:]) + b_ref[:]
    h = jnp.maximum(z, 0.0)
    h2 = jnp.dot(h, w_ref[:], preferred_element_type=jnp.float32)
    o_ref[:] = h2 * dis


def _tc3_body(p0_ref, p1_ref, hp_ref, dis_ref, b_ref, o_ref):
    o_ref[:] = dis_ref[:] * (p0_ref[:] + p1_ref[:] + hp_ref[:]) + b_ref[:]


_row_spec = pl.BlockSpec((R_BLK, D), lambda i: (i, 0))
_w_spec = pl.BlockSpec((D, D), lambda i: (0, 0))
_b_spec = pl.BlockSpec((1, D), lambda i: (0, 0))
_out_sd = jax.ShapeDtypeStruct((N, D), jnp.float32)

_tc1 = pl.pallas_call(
    _tc1_body, grid=(GRID,),
    in_specs=[_row_spec, _w_spec, _row_spec, _row_spec],
    out_specs=[_row_spec, _row_spec], out_shape=[_out_sd, _out_sd])

_tc2 = pl.pallas_call(
    _tc2_body, grid=(GRID,),
    in_specs=[_row_spec, _row_spec, _row_spec, _row_spec, _b_spec, _w_spec],
    out_specs=_row_spec, out_shape=_out_sd)

_tc3 = pl.pallas_call(
    _tc3_body, grid=(GRID,),
    in_specs=[_row_spec, _row_spec, _row_spec, _row_spec, _b_spec],
    out_specs=_row_spec, out_shape=_out_sd)


def kernel(x, edge_index, W1, b1, W2, b2):
    src = edge_index[0]
    dst = edge_index[1]
    e_alloc = E_PAD + (NBUF - 1) * AEB
    src_p = jnp.concatenate([src, jnp.zeros((e_alloc - E,), jnp.int32)])
    dst_p = jnp.concatenate([dst, jnp.full((e_alloc - E,), N, jnp.int32)])
    pairs = jnp.stack(
        [src_p.reshape(-1, AEB), dst_p.reshape(-1, AEB)], axis=1
    ).reshape(-1, AEB)

    ones128 = jnp.zeros((EB, D), jnp.float32).at[:, 0].set(1.0)
    zeros128 = jnp.zeros((EB, D), jnp.float32)
    zeros_a = jnp.zeros((AEB, D), jnp.float32)

    degc = _deg_call()(dst_p[:E_PAD], ones128, zeros128)

    b1r = b1.reshape(1, D)
    b2r = b2.reshape(1, D)

    h1p, dis = _tc1(x, W1, degc[:N], degc[N_PAD:N_PAD + N])
    s1 = _agg_call()(h1p, pairs, zeros_a)
    h2p = _tc2(s1[:N], s1[N_PAD:N_PAD + N], h1p, dis, b1r, W2)
    s2 = _agg_call()(h2p, pairs, zeros_a)
    return _tc3(s2[:N], s2[N_PAD:N_PAD + N], h2p, dis, b2r)


# speedup vs baseline: 1.2534x; 1.0081x over previous
import functools

import jax
import jax.numpy as jnp
from jax import lax
from jax.experimental import pallas as pl
from jax.experimental.pallas import tpu as pltpu
from jax.experimental.pallas import tpu_sc as plsc

N = 10000
D = 128
E = 320000

NC = 2
NS = 16
NW = NC * NS
EB = 128
BLOCKS = 80
E_PAD = NW * BLOCKS * EB
E_PER_SC = E_PAD // NC
E_PER_TILE = E_PER_SC // NS
N_PAD = 10240
ROWS_PER_TILE = N_PAD // NS
STAGE_ROWS = 128
N_STAGE = ROWS_PER_TILE // STAGE_ROWS

AEB = 128
NBUF = 2
ABLK0 = 128
ABLK1 = 160 - ABLK0
A_N_STAGE = ROWS_PER_TILE // AEB


def _deg_body(dst_hbm, ones_hbm, zeros_hbm, out_hbm, didx, crow, acc):
    c = lax.axis_index("c")
    s = lax.axis_index("s")
    pltpu.sync_copy(zeros_hbm, crow)
    for k in range(N_STAGE):
        pltpu.sync_copy(
            crow, acc.at[pl.ds(s * ROWS_PER_TILE + k * STAGE_ROWS, STAGE_ROWS)]
        )
    pltpu.sync_copy(ones_hbm, crow)
    plsc.subcore_barrier()

    tile_base = c * E_PER_SC + s * E_PER_TILE

    def body(i, carry):
        base = tile_base + i * EB
        pltpu.sync_copy(dst_hbm.at[pl.ds(base, EB)], didx)
        pltpu.sync_copy(crow, acc.at[didx], add=True)
        return carry

    lax.fori_loop(0, BLOCKS, body, 0)
    plsc.subcore_barrier()
    for k in range(N_STAGE):
        r0 = s * ROWS_PER_TILE + k * STAGE_ROWS
        pltpu.sync_copy(acc.at[pl.ds(r0, STAGE_ROWS)], crow)
        pltpu.sync_copy(crow, out_hbm.at[pl.ds(c * N_PAD + r0, STAGE_ROWS)])


@functools.cache
def _deg_call():
    return pl.kernel(
        _deg_body,
        mesh=plsc.VectorSubcoreMesh(core_axis_name="c", subcore_axis_name="s"),
        out_type=jax.ShapeDtypeStruct((2 * N_PAD, D), jnp.float32),
        scratch_types=[
            pltpu.VMEM((EB,), jnp.int32),
            pltpu.VMEM((EB, D), jnp.float32),
            pltpu.VMEM_SHARED((N_PAD, D), jnp.float32),
        ],
    )


def _agg_body(hp_hbm, pairs_hbm, zeros_hbm, out_hbm, *sc):
    pidx = sc[:NBUF]
    rows = sc[NBUF:2 * NBUF]
    acc = sc[2 * NBUF]
    sems = sc[2 * NBUF + 1:]
    c = lax.axis_index("c")
    s = lax.axis_index("s")

    pltpu.sync_copy(zeros_hbm, rows[0])
    for k in range(A_N_STAGE):
        pltpu.sync_copy(
            rows[0], acc.at[pl.ds(s * ROWS_PER_TILE + k * AEB, AEB)]
        )
    plsc.subcore_barrier()

    tile_blk0 = jnp.where(c == 0, s * ABLK0, NS * ABLK0 + s * ABLK1)
    n_outer = jnp.where(c == 0, ABLK0 // NBUF, ABLK1 // NBUF)

    def fire(blk, b):
        pltpu.sync_copy(pairs_hbm.at[pl.ds(2 * blk, 2)], pidx[b])
        pltpu.async_copy(hp_hbm.at[pidx[b].at[0]], rows[b], sems[b])

    def drain(b):
        pltpu.make_async_copy(hp_hbm.at[pidx[b].at[0]], rows[b], sems[b]).wait()

    for b in range(NBUF - 1):
        fire(tile_blk0 + b, b)

    def body(g, carry):
        i0 = tile_blk0 + NBUF * g
        for b in range(NBUF):
            fire(i0 + b + NBUF - 1, (b + NBUF - 1) % NBUF)
            drain(b)
            pltpu.sync_copy(rows[b], acc.at[pidx[b].at[1]], add=True)
        return carry

    lax.fori_loop(0, n_outer, body, 0)
    for b in range(NBUF - 1):
        drain(b)
    plsc.subcore_barrier()
    for k in range(A_N_STAGE):
        r0 = s * ROWS_PER_TILE + k * AEB
        pltpu.sync_copy(acc.at[pl.ds(r0, AEB)], rows[0])
        pltpu.sync_copy(rows[0], out_hbm.at[pl.ds(c * N_PAD + r0, AEB)])


@functools.cache
def _agg_call():
    return pl.kernel(
        _agg_body,
        mesh=plsc.VectorSubcoreMesh(core_axis_name="c", subcore_axis_name="s"),
        out_type=jax.ShapeDtypeStruct((2 * N_PAD, D), jnp.float32),
        scratch_types=(
            [pltpu.VMEM((2, AEB), jnp.int32) for _ in range(NBUF)]
            + [pltpu.VMEM((AEB, D), jnp.float32) for _ in range(NBUF)]
            + [pltpu.VMEM_SHARED((N_PAD, D), jnp.float32)]
            + [pltpu.SemaphoreType.DMA for _ in range(NBUF)]
        ),
    )


R_BLK = 2000
GRID = N // R_BLK


def _tc1_body(x_ref, w_ref, d0_ref, d1_ref, o_ref, dis_ref):
    deg = d0_ref[:, 0:1] + d1_ref[:, 0:1] + 1.0
    dis = lax.rsqrt(deg)
    h = jnp.dot(x_ref[:], w_ref[:], preferred_element_type=jnp.float32)
    o_ref[:] = h * dis
    dis_ref[:] = jnp.broadcast_to(dis, (R_BLK, D))


def _tc2_body(p0_ref, p1_ref, hp_ref, dis_ref, b_ref, w_ref, o_ref):
    dis = dis_ref[:]
    z = dis * (p0_ref[:] + p1_ref[:] + hp_ref[:]) + b_ref[:]
    h = jnp.maximum(z, 0.0)
    h2 = jnp.dot(h, w_ref[:], preferred_element_type=jnp.float32)
    o_ref[:] = h2 * dis


def _tc3_body(p0_ref, p1_ref, hp_ref, dis_ref, b_ref, o_ref):
    o_ref[:] = dis_ref[:] * (p0_ref[:] + p1_ref[:] + hp_ref[:]) + b_ref[:]


_row_spec = pl.BlockSpec((R_BLK, D), lambda i: (i, 0))
_w_spec = pl.BlockSpec((D, D), lambda i: (0, 0))
_b_spec = pl.BlockSpec((1, D), lambda i: (0, 0))
_out_sd = jax.ShapeDtypeStruct((N, D), jnp.float32)

_tc1 = pl.pallas_call(
    _tc1_body, grid=(GRID,),
    in_specs=[_row_spec, _w_spec, _row_spec, _row_spec],
    out_specs=[_row_spec, _row_spec], out_shape=[_out_sd, _out_sd])

_tc2 = pl.pallas_call(
    _tc2_body, grid=(GRID,),
    in_specs=[_row_spec, _row_spec, _row_spec, _row_spec, _b_spec, _w_spec],
    out_specs=_row_spec, out_shape=_out_sd)

_tc3 = pl.pallas_call(
    _tc3_body, grid=(GRID,),
    in_specs=[_row_spec, _row_spec, _row_spec, _row_spec, _b_spec],
    out_specs=_row_spec, out_shape=_out_sd)


def kernel(x, edge_index, W1, b1, W2, b2):
    src = edge_index[0]
    dst = edge_index[1]
    e_alloc = E_PAD + (NBUF - 1) * AEB
    src_p = jnp.concatenate([src, jnp.zeros((e_alloc - E,), jnp.int32)])
    dst_p = jnp.concatenate([dst, jnp.full((e_alloc - E,), N, jnp.int32)])
    pairs = jnp.stack(
        [src_p.reshape(-1, AEB), dst_p.reshape(-1, AEB)], axis=1
    ).reshape(-1, AEB)

    ones128 = jnp.zeros((EB, D), jnp.float32).at[:, 0].set(1.0)
    zeros128 = jnp.zeros((EB, D), jnp.float32)
    zeros_a = jnp.zeros((AEB, D), jnp.float32)

    degc = _deg_call()(dst_p[:E_PAD], ones128, zeros128)

    b1r = b1.reshape(1, D)
    b2r = b2.reshape(1, D)

    h1p, dis = _tc1(x, W1, degc[:N], degc[N_PAD:N_PAD + N])
    s1 = _agg_call()(h1p, pairs, zeros_a)
    h2p = _tc2(s1[:N], s1[N_PAD:N_PAD + N], h1p, dis, b1r, W2)
    s2 = _agg_call()(h2p, pairs, zeros_a)
    return _tc3(s2[:N], s2[N_PAD:N_PAD + N], h2p, dis, b2r)


# speedup vs baseline: 1.2679x; 1.0116x over previous
import functools

import jax
import jax.numpy as jnp
from jax import lax
from jax.experimental import pallas as pl
from jax.experimental.pallas import tpu as pltpu
from jax.experimental.pallas import tpu_sc as plsc

N = 10000
D = 128
E = 320000

NC = 2
NS = 16
NW = NC * NS
EB = 128
BLOCKS = 80
E_PAD = NW * BLOCKS * EB
E_PER_SC = E_PAD // NC
E_PER_TILE = E_PER_SC // NS
N_PAD = 10240
ROWS_PER_TILE = N_PAD // NS
STAGE_ROWS = 128
N_STAGE = ROWS_PER_TILE // STAGE_ROWS

AEB = 128
NBUF = 2
ABLK0 = 136
ABLK1 = 160 - ABLK0
A_N_STAGE = ROWS_PER_TILE // AEB


def _deg_body(dst_hbm, ones_hbm, zeros_hbm, out_hbm, didx, crow, acc):
    c = lax.axis_index("c")
    s = lax.axis_index("s")
    pltpu.sync_copy(zeros_hbm, crow)
    for k in range(N_STAGE):
        pltpu.sync_copy(
            crow, acc.at[pl.ds(s * ROWS_PER_TILE + k * STAGE_ROWS, STAGE_ROWS)]
        )
    pltpu.sync_copy(ones_hbm, crow)
    plsc.subcore_barrier()

    tile_base = c * E_PER_SC + s * E_PER_TILE

    def body(i, carry):
        base = tile_base + i * EB
        pltpu.sync_copy(dst_hbm.at[pl.ds(base, EB)], didx)
        pltpu.sync_copy(crow, acc.at[didx], add=True)
        return carry

    lax.fori_loop(0, BLOCKS, body, 0)
    plsc.subcore_barrier()
    for k in range(N_STAGE):
        r0 = s * ROWS_PER_TILE + k * STAGE_ROWS
        pltpu.sync_copy(acc.at[pl.ds(r0, STAGE_ROWS)], crow)
        pltpu.sync_copy(crow, out_hbm.at[pl.ds(c * N_PAD + r0, STAGE_ROWS)])


@functools.cache
def _deg_call():
    return pl.kernel(
        _deg_body,
        mesh=plsc.VectorSubcoreMesh(core_axis_name="c", subcore_axis_name="s"),
        out_type=jax.ShapeDtypeStruct((2 * N_PAD, D), jnp.float32),
        scratch_types=[
            pltpu.VMEM((EB,), jnp.int32),
            pltpu.VMEM((EB, D), jnp.float32),
            pltpu.VMEM_SHARED((N_PAD, D), jnp.float32),
        ],
    )


def _agg_body(hp_hbm, pairs_hbm, zeros_hbm, out_hbm, *sc):
    pidx = sc[:NBUF]
    rows = sc[NBUF:2 * NBUF]
    acc = sc[2 * NBUF]
    sems = sc[2 * NBUF + 1:]
    c = lax.axis_index("c")
    s = lax.axis_index("s")

    pltpu.sync_copy(zeros_hbm, rows[0])
    for k in range(A_N_STAGE):
        pltpu.sync_copy(
            rows[0], acc.at[pl.ds(s * ROWS_PER_TILE + k * AEB, AEB)]
        )
    plsc.subcore_barrier()

    tile_blk0 = jnp.where(c == 0, s * ABLK0, NS * ABLK0 + s * ABLK1)
    n_outer = jnp.where(c == 0, ABLK0 // NBUF, ABLK1 // NBUF)

    def fire(blk, b):
        pltpu.sync_copy(pairs_hbm.at[pl.ds(2 * blk, 2)], pidx[b])
        pltpu.async_copy(hp_hbm.at[pidx[b].at[0]], rows[b], sems[b])

    def drain(b):
        pltpu.make_async_copy(hp_hbm.at[pidx[b].at[0]], rows[b], sems[b]).wait()

    for b in range(NBUF - 1):
        fire(tile_blk0 + b, b)

    def body(g, carry):
        i0 = tile_blk0 + NBUF * g
        for b in range(NBUF):
            fire(i0 + b + NBUF - 1, (b + NBUF - 1) % NBUF)
            drain(b)
            pltpu.sync_copy(rows[b], acc.at[pidx[b].at[1]], add=True)
        return carry

    lax.fori_loop(0, n_outer, body, 0)
    for b in range(NBUF - 1):
        drain(b)
    plsc.subcore_barrier()
    for k in range(A_N_STAGE):
        r0 = s * ROWS_PER_TILE + k * AEB
        pltpu.sync_copy(acc.at[pl.ds(r0, AEB)], rows[0])
        pltpu.sync_copy(rows[0], out_hbm.at[pl.ds(c * N_PAD + r0, AEB)])


@functools.cache
def _agg_call():
    return pl.kernel(
        _agg_body,
        mesh=plsc.VectorSubcoreMesh(core_axis_name="c", subcore_axis_name="s"),
        out_type=jax.ShapeDtypeStruct((2 * N_PAD, D), jnp.float32),
        scratch_types=(
            [pltpu.VMEM((2, AEB), jnp.int32) for _ in range(NBUF)]
            + [pltpu.VMEM((AEB, D), jnp.float32) for _ in range(NBUF)]
            + [pltpu.VMEM_SHARED((N_PAD, D), jnp.float32)]
            + [pltpu.SemaphoreType.DMA for _ in range(NBUF)]
        ),
    )


R_BLK = 2000
GRID = N // R_BLK


def _tc1_body(x_ref, w_ref, d0_ref, d1_ref, o_ref, dis_ref):
    deg = d0_ref[:, 0:1] + d1_ref[:, 0:1] + 1.0
    dis = lax.rsqrt(deg)
    h = jnp.dot(x_ref[:], w_ref[:], preferred_element_type=jnp.float32)
    o_ref[:] = h * dis
    dis_ref[:] = jnp.broadcast_to(dis, (R_BLK, D))


def _tc2_body(p0_ref, p1_ref, hp_ref, dis_ref, b_ref, w_ref, o_ref):
    dis = dis_ref[:]
    z = dis * (p0_ref[:] + p1_ref[:] + hp_ref[:]) + b_ref[:]
    h = jnp.maximum(z, 0.0)
    h2 = jnp.dot(h, w_ref[:], preferred_element_type=jnp.float32)
    o_ref[:] = h2 * dis


def _tc3_body(p0_ref, p1_ref, hp_ref, dis_ref, b_ref, o_ref):
    o_ref[:] = dis_ref[:] * (p0_ref[:] + p1_ref[:] + hp_ref[:]) + b_ref[:]


_row_spec = pl.BlockSpec((R_BLK, D), lambda i: (i, 0))
_w_spec = pl.BlockSpec((D, D), lambda i: (0, 0))
_b_spec = pl.BlockSpec((1, D), lambda i: (0, 0))
_out_sd = jax.ShapeDtypeStruct((N, D), jnp.float32)

_tc1 = pl.pallas_call(
    _tc1_body, grid=(GRID,),
    in_specs=[_row_spec, _w_spec, _row_spec, _row_spec],
    out_specs=[_row_spec, _row_spec], out_shape=[_out_sd, _out_sd])

_tc2 = pl.pallas_call(
    _tc2_body, grid=(GRID,),
    in_specs=[_row_spec, _row_spec, _row_spec, _row_spec, _b_spec, _w_spec],
    out_specs=_row_spec, out_shape=_out_sd)

_tc3 = pl.pallas_call(
    _tc3_body, grid=(GRID,),
    in_specs=[_row_spec, _row_spec, _row_spec, _row_spec, _b_spec],
    out_specs=_row_spec, out_shape=_out_sd)


def kernel(x, edge_index, W1, b1, W2, b2):
    src = edge_index[0]
    dst = edge_index[1]
    e_alloc = E_PAD + (NBUF - 1) * AEB
    src_p = jnp.concatenate([src, jnp.zeros((e_alloc - E,), jnp.int32)])
    dst_p = jnp.concatenate([dst, jnp.full((e_alloc - E,), N, jnp.int32)])
    pairs = jnp.stack(
        [src_p.reshape(-1, AEB), dst_p.reshape(-1, AEB)], axis=1
    ).reshape(-1, AEB)

    ones128 = jnp.zeros((EB, D), jnp.float32).at[:, 0].set(1.0)
    zeros128 = jnp.zeros((EB, D), jnp.float32)
    zeros_a = jnp.zeros((AEB, D), jnp.float32)

    degc = _deg_call()(dst_p[:E_PAD], ones128, zeros128)

    b1r = b1.reshape(1, D)
    b2r = b2.reshape(1, D)

    h1p, dis = _tc1(x, W1, degc[:N], degc[N_PAD:N_PAD + N])
    s1 = _agg_call()(h1p, pairs, zeros_a)
    h2p = _tc2(s1[:N], s1[N_PAD:N_PAD + N], h1p, dis, b1r, W2)
    s2 = _agg_call()(h2p, pairs, zeros_a)
    return _tc3(s2[:N], s2[N_PAD:N_PAD + N], h2p, dis, b2r)
